# baseline (speedup 1.0000x reference)
import functools

import jax
import jax.numpy as jnp
from jax import lax
from jax.experimental import pallas as pl
from jax.experimental.pallas import tpu as pltpu
from jax.experimental.pallas import tpu_sc as plsc

NC = 2
NS = 16
NW = NC * NS
LANES = 16
C = 64
NBUF = 3


def _sqnorm_tc(h):
    n_nodes, d_feat = h.shape
    blk = 2000
    assert n_nodes % blk == 0

    def body(h_ref, o_ref):
        x = h_ref[...]
        o_ref[...] = jnp.sum(x * x, axis=1, keepdims=True)

    return pl.pallas_call(
        body,
        grid=(n_nodes // blk,),
        in_specs=[pl.BlockSpec((blk, d_feat), lambda i: (i, 0))],
        out_specs=pl.BlockSpec((blk, 1), lambda i: (i, 0)),
        out_shape=jax.ShapeDtypeStruct((n_nodes, 1), jnp.float32),
    )(h)


def _dot_chunk(rows_w, sq, idx_s, idx_d, off, out_c, d_feat):
    n_seg = d_feat // LANES
    lanes = lax.iota(jnp.int32, LANES)
    gdn = lax.GatherDimensionNumbers(
        offset_dims=(), collapsed_slice_dims=(0,), start_index_map=(0,)
    )
    perm_idx = [((lanes ^ (1 << k))[:, None]) for k in range(4)]
    masks = [(lanes & (1 << k)) == 0 for k in range(4)]

    def perm(x, k):
        return lax.gather(
            x, perm_idx[k], gdn, (1,),
            mode=lax.GatherScatterMode.PROMISE_IN_BOUNDS,
        )

    def combine(a, b, k):
        return jnp.where(masks[k], a + perm(a, k), b + perm(b, k))

    def group_body(g, _):
        levels = [None] * 4
        node = None
        for i in range(LANES):
            e = g * LANES + i
            node = None
            for j in range(n_seg):
                w = rows_w[e, pl.ds(j * LANES, LANES)]
                t = w * w
                node = t if node is None else node + t
            for k in range(4):
                if levels[k] is None:
                    levels[k] = node
                    node = None
                    break
                node = combine(levels[k], node, k)
                levels[k] = None
        ebase = pl.multiple_of(off + g * LANES, LANES)
        su = plsc.load_gather(sq, [idx_s[pl.ds(ebase, LANES)]])
        sv = plsc.load_gather(sq, [idx_d[pl.ds(ebase, LANES)]])
        out_c[pl.ds(g * LANES, LANES)] = (node - su - sv) * 0.5
        return 0

    lax.fori_loop(0, C // LANES, group_body, 0)


def _sc_dot(h, sq, src, dst):
    n_nodes, d_feat = h.shape
    n_edges = src.shape[0]
    n_chunks = n_edges // C
    base_rounds = n_chunks // NW
    tail = n_chunks - base_rounds * NW
    assert base_rounds % NBUF == 0 and base_rounds >= 2 * NBUF

    mesh = plsc.VectorSubcoreMesh(
        core_axis_name="c", subcore_axis_name="s", num_cores=NC, num_subcores=NS
    )

    @functools.partial(
        pl.kernel,
        out_type=jax.ShapeDtypeStruct((n_edges,), jnp.float32),
        mesh=mesh,
        compiler_params=pltpu.CompilerParams(needs_layout_passes=False),
        scratch_types=[
            pltpu.VMEM(((base_rounds + (1 if tail else 0)) * C,), jnp.int32),
            pltpu.VMEM(((base_rounds + (1 if tail else 0)) * C,), jnp.int32),
            pltpu.VMEM((NBUF, C, d_feat), jnp.float32),
            pltpu.VMEM((n_nodes,), jnp.float32),
            pltpu.VMEM((2, C), jnp.float32),
            pltpu.SemaphoreType.DMA,
            pltpu.SemaphoreType.DMA,
            pltpu.SemaphoreType.DMA,
            pltpu.SemaphoreType.DMA,
            pltpu.SemaphoreType.DMA,
        ],
    )
    def k(h_hbm, sq_hbm, src_hbm, dst_hbm, out_hbm,
          idx_s, idx_d, rows_w, sq_v, out_ring,
          sem_w0, sem_w1, sem_w2, sem_o0, sem_o1):
        wid = lax.axis_index("s") * NC + lax.axis_index("c")
        extra = wid < tail
        n_w = base_rounds + extra.astype(jnp.int32)
        s_w = wid * base_rounds + jnp.minimum(wid, tail)
        e_w = pl.multiple_of(s_w * C, C)
        n_base = base_rounds * C
        sems_w = (sem_w0, sem_w1, sem_w2)
        sems_o = (sem_o0, sem_o1)

        pltpu.async_copy(sq_hbm.at[pl.ds(0, n_nodes)], sq_v, sem_w0)
        pltpu.async_copy(src_hbm.at[pl.ds(e_w, n_base)], idx_s.at[pl.ds(0, n_base)], sem_w1)
        pltpu.async_copy(dst_hbm.at[pl.ds(e_w, n_base)], idx_d.at[pl.ds(0, n_base)], sem_w2)
        pltpu.make_async_copy(sq_hbm.at[pl.ds(0, n_nodes)], sq_v, sem_w0).wait()
        pltpu.make_async_copy(src_hbm.at[pl.ds(e_w, n_base)], idx_s.at[pl.ds(0, n_base)], sem_w1).wait()
        pltpu.make_async_copy(dst_hbm.at[pl.ds(e_w, n_base)], idx_d.at[pl.ds(0, n_base)], sem_w2).wait()
        if tail:
            @pl.when(extra)
            def _():
                pltpu.sync_copy(src_hbm.at[pl.ds(e_w + n_base, C)],
                                idx_s.at[pl.ds(n_base, C)])
                pltpu.sync_copy(dst_hbm.at[pl.ds(e_w + n_base, C)],
                                idx_d.at[pl.ds(n_base, C)])

        def start_p1(i, b):
            off = pl.multiple_of(i * C, C)
            pltpu.async_copy(h_hbm.at[idx_s.at[pl.ds(off, C)]], rows_w.at[b], sems_w[b])

        def start_p2(i, b):
            off = pl.multiple_of(i * C, C)
            pltpu.make_async_copy(h_hbm.at[idx_s.at[pl.ds(off, C)]], rows_w.at[b], sems_w[b]).wait()
            pltpu.async_copy(h_hbm.at[idx_d.at[pl.ds(off, C)]], rows_w.at[b], sems_w[b], add=True)

        def compute(i, b, ob):
            off = pl.multiple_of(i * C, C)
            pltpu.make_async_copy(h_hbm.at[idx_d.at[pl.ds(off, C)]], rows_w.at[b], sems_w[b]).wait()

            @pl.when(i >= 2)
            def _():
                pltpu.make_async_copy(out_ring.at[ob], out_hbm.at[pl.ds(e_w, C)], sems_o[ob]).wait()

            _dot_chunk(rows_w.at[b], sq_v, idx_s, idx_d, off, out_ring.at[ob], d_feat)
            pltpu.async_copy(out_ring.at[ob], out_hbm.at[pl.ds(e_w + off, C)], sems_o[ob])

        start_p1(0, 0)
        start_p1(1, 1)
        start_p2(0, 0)

        def step(i, b, ob):
            @pl.when(i + 2 < n_w)
            def _():
                start_p1(i + 2, (b + 2) % NBUF)

            @pl.when(i + 1 < n_w)
            def _():
                start_p2(i + 1, (b + 1) % NBUF)

            compute(i, b, ob)

        def loop_body(r, _):
            i = r * NBUF
            for u in range(NBUF):
                step(i + u, u, 0 if u % 2 == 0 else 1)
            return 0

        lax.fori_loop(0, base_rounds // NBUF, loop_body, 0)

        if tail:
            @pl.when(extra)
            def _():
                compute(base_rounds, base_rounds % NBUF, base_rounds % 2)

        pltpu.make_async_copy(out_ring.at[0], out_hbm.at[pl.ds(e_w, C)], sem_o0).wait()
        pltpu.make_async_copy(out_ring.at[1], out_hbm.at[pl.ds(e_w, C)], sem_o1).wait()

    return k(h, sq, src, dst)


@jax.jit
def kernel(h, edge_index):
    sq = _sqnorm_tc(h).reshape(-1)
    score = _sc_dot(h, sq, edge_index[0], edge_index[1])
    return score[:, None]


# speedup vs baseline: 1.1135x; 1.1135x over previous
import functools

import jax
import jax.numpy as jnp
from jax import lax
from jax.experimental import pallas as pl
from jax.experimental.pallas import tpu as pltpu
from jax.experimental.pallas import tpu_sc as plsc

NC = 2
NS = 16
NW = NC * NS
LANES = 16
C = 128
NBUF = 3


def _sqnorm_tc(h):
    n_nodes, d_feat = h.shape
    blk = 2000
    assert n_nodes % blk == 0

    def body(h_ref, o_ref):
        x = h_ref[...]
        o_ref[...] = jnp.sum(x * x, axis=1, keepdims=True)

    return pl.pallas_call(
        body,
        grid=(n_nodes // blk,),
        in_specs=[pl.BlockSpec((blk, d_feat), lambda i: (i, 0))],
        out_specs=pl.BlockSpec((blk, 1), lambda i: (i, 0)),
        out_shape=jax.ShapeDtypeStruct((n_nodes, 1), jnp.float32),
    )(h)


def _dot_chunk(rows_w, sq, idx_s, idx_d, off, out_c, d_feat):
    n_seg = d_feat // LANES
    lanes = lax.iota(jnp.int32, LANES)
    gdn = lax.GatherDimensionNumbers(
        offset_dims=(), collapsed_slice_dims=(0,), start_index_map=(0,)
    )
    perm_idx = [((lanes ^ (1 << k))[:, None]) for k in range(4)]
    masks = [(lanes & (1 << k)) == 0 for k in range(4)]

    def perm(x, k):
        return lax.gather(
            x, perm_idx[k], gdn, (1,),
            mode=lax.GatherScatterMode.PROMISE_IN_BOUNDS,
        )

    def combine(a, b, k):
        return jnp.where(masks[k], a + perm(a, k), b + perm(b, k))

    def group_body(g, _):
        levels = [None] * 4
        node = None
        for i in range(LANES):
            e = g * LANES + i
            node = None
            for j in range(n_seg):
                w = rows_w[e, pl.ds(j * LANES, LANES)]
                t = w * w
                node = t if node is None else node + t
            for k in range(4):
                if levels[k] is None:
                    levels[k] = node
                    node = None
                    break
                node = combine(levels[k], node, k)
                levels[k] = None
        ebase = pl.multiple_of(off + g * LANES, LANES)
        su = plsc.load_gather(sq, [idx_s[pl.ds(ebase, LANES)]])
        sv = plsc.load_gather(sq, [idx_d[pl.ds(ebase, LANES)]])
        out_c[pl.ds(g * LANES, LANES)] = (node - su - sv) * 0.5
        return 0

    lax.fori_loop(0, C // LANES, group_body, 0)


def _sc_dot(h, sq, src, dst):
    n_nodes, d_feat = h.shape
    n_edges = src.shape[0]
    n_chunks = n_edges // C
    base_rounds = n_chunks // NW
    tail = n_chunks - base_rounds * NW
    assert base_rounds % NBUF == 0 and base_rounds >= 2 * NBUF

    mesh = plsc.VectorSubcoreMesh(
        core_axis_name="c", subcore_axis_name="s", num_cores=NC, num_subcores=NS
    )

    @functools.partial(
        pl.kernel,
        out_type=jax.ShapeDtypeStruct((n_edges,), jnp.float32),
        mesh=mesh,
        compiler_params=pltpu.CompilerParams(needs_layout_passes=False),
        scratch_types=[
            pltpu.VMEM(((base_rounds + (1 if tail else 0)) * C,), jnp.int32),
            pltpu.VMEM(((base_rounds + (1 if tail else 0)) * C,), jnp.int32),
            pltpu.VMEM((NBUF, C, d_feat), jnp.float32),
            pltpu.VMEM((n_nodes,), jnp.float32),
            pltpu.VMEM((2, C), jnp.float32),
            pltpu.SemaphoreType.DMA,
            pltpu.SemaphoreType.DMA,
            pltpu.SemaphoreType.DMA,
            pltpu.SemaphoreType.DMA,
            pltpu.SemaphoreType.DMA,
        ],
    )
    def k(h_hbm, sq_hbm, src_hbm, dst_hbm, out_hbm,
          idx_s, idx_d, rows_w, sq_v, out_ring,
          sem_w0, sem_w1, sem_w2, sem_o0, sem_o1):
        wid = lax.axis_index("s") * NC + lax.axis_index("c")
        extra = wid < tail
        n_w = base_rounds + extra.astype(jnp.int32)
        s_w = wid * base_rounds + jnp.minimum(wid, tail)
        e_w = pl.multiple_of(s_w * C, C)
        n_base = base_rounds * C
        sems_w = (sem_w0, sem_w1, sem_w2)
        sems_o = (sem_o0, sem_o1)

        pltpu.async_copy(sq_hbm.at[pl.ds(0, n_nodes)], sq_v, sem_w0)
        pltpu.async_copy(src_hbm.at[pl.ds(e_w, n_base)], idx_s.at[pl.ds(0, n_base)], sem_w1)
        pltpu.async_copy(dst_hbm.at[pl.ds(e_w, n_base)], idx_d.at[pl.ds(0, n_base)], sem_w2)
        pltpu.make_async_copy(sq_hbm.at[pl.ds(0, n_nodes)], sq_v, sem_w0).wait()
        pltpu.make_async_copy(src_hbm.at[pl.ds(e_w, n_base)], idx_s.at[pl.ds(0, n_base)], sem_w1).wait()
        pltpu.make_async_copy(dst_hbm.at[pl.ds(e_w, n_base)], idx_d.at[pl.ds(0, n_base)], sem_w2).wait()
        if tail:
            @pl.when(extra)
            def _():
                pltpu.sync_copy(src_hbm.at[pl.ds(e_w + n_base, C)],
                                idx_s.at[pl.ds(n_base, C)])
                pltpu.sync_copy(dst_hbm.at[pl.ds(e_w + n_base, C)],
                                idx_d.at[pl.ds(n_base, C)])

        def start_p1(i, b):
            off = pl.multiple_of(i * C, C)
            pltpu.async_copy(h_hbm.at[idx_s.at[pl.ds(off, C)]], rows_w.at[b], sems_w[b])

        def start_p2(i, b):
            off = pl.multiple_of(i * C, C)
            pltpu.make_async_copy(h_hbm.at[idx_s.at[pl.ds(off, C)]], rows_w.at[b], sems_w[b]).wait()
            pltpu.async_copy(h_hbm.at[idx_d.at[pl.ds(off, C)]], rows_w.at[b], sems_w[b], add=True)

        def compute(i, b, ob):
            off = pl.multiple_of(i * C, C)
            pltpu.make_async_copy(h_hbm.at[idx_d.at[pl.ds(off, C)]], rows_w.at[b], sems_w[b]).wait()

            @pl.when(i >= 2)
            def _():
                pltpu.make_async_copy(out_ring.at[ob], out_hbm.at[pl.ds(e_w, C)], sems_o[ob]).wait()

            _dot_chunk(rows_w.at[b], sq_v, idx_s, idx_d, off, out_ring.at[ob], d_feat)
            pltpu.async_copy(out_ring.at[ob], out_hbm.at[pl.ds(e_w + off, C)], sems_o[ob])

        start_p1(0, 0)
        start_p1(1, 1)
        start_p2(0, 0)

        def step(i, b, ob):
            @pl.when(i + 2 < n_w)
            def _():
                start_p1(i + 2, (b + 2) % NBUF)

            @pl.when(i + 1 < n_w)
            def _():
                start_p2(i + 1, (b + 1) % NBUF)

            compute(i, b, ob)

        def loop_body(r, _):
            i = r * NBUF
            for u in range(NBUF):
                step(i + u, u, 0 if u % 2 == 0 else 1)
            return 0

        lax.fori_loop(0, base_rounds // NBUF, loop_body, 0)

        if tail:
            @pl.when(extra)
            def _():
                compute(base_rounds, base_rounds % NBUF, base_rounds % 2)

        pltpu.make_async_copy(out_ring.at[0], out_hbm.at[pl.ds(e_w, C)], sem_o0).wait()
        pltpu.make_async_copy(out_ring.at[1], out_hbm.at[pl.ds(e_w, C)], sem_o1).wait()

    return k(h, sq, src, dst)


@jax.jit
def kernel(h, edge_index):
    sq = _sqnorm_tc(h).reshape(-1)
    score = _sc_dot(h, sq, edge_index[0], edge_index[1])
    return score[:, None]
